# Initial kernel scaffold; baseline (speedup 1.0000x reference)
import jax
import jax.numpy as jnp
from jax.experimental import pallas as pl


def kernel(x, edge_index):
    raise NotImplementedError("write your pallas kernel here")


# speedup vs baseline: 3.4474x; 3.4474x over previous
import functools

import jax
import jax.numpy as jnp
from jax import lax
from jax.experimental import pallas as pl
from jax.experimental.pallas import tpu as pltpu
from jax.experimental.pallas import tpu_sc as plsc

_N = 10000
_E = 320000
_D = 128
_NC = 2
_NS = 16
_HALF = _D // _NC
_EPT = _E // _NS
_CHUNK = 80
_NCHUNK = _EPT // _CHUNK
_RPT = 624
_RREM = _N - _NS * _RPT


def _sc_body(x2, src, dst, zero, out, sidx, didx, rows, acc, sem):
    c = lax.axis_index("c")
    s = lax.axis_index("s")
    r0 = s * _RPT

    pltpu.sync_copy(zero.at[pl.ds(r0, _RPT)], acc.at[pl.ds(r0, _RPT)])

    @pl.when(s == 0)
    def _zero_rem():
        pltpu.sync_copy(
            zero.at[pl.ds(_NS * _RPT, _RREM)], acc.at[pl.ds(_NS * _RPT, _RREM)]
        )

    plsc.subcore_barrier()

    base0 = s * _EPT
    coff = c * _N

    def body(i, carry):
        base = base0 + i * _CHUNK
        pltpu.sync_copy(src.at[pl.ds(base, _CHUNK)], sidx)
        pltpu.sync_copy(dst.at[pl.ds(base, _CHUNK)], didx)
        for j in range(_CHUNK // 16):
            sl = pl.ds(j * 16, 16)
            sidx[sl] = sidx[sl] + coff
        pltpu.async_copy(x2.at[sidx], rows, sem).wait()
        pltpu.sync_copy(rows, acc.at[didx], add=True)
        return carry

    lax.fori_loop(0, _NCHUNK, body, 0)
    plsc.subcore_barrier()

    pltpu.sync_copy(acc.at[pl.ds(r0, _RPT)], out.at[c].at[pl.ds(r0, _RPT)])

    @pl.when(s == 0)
    def _write_rem():
        pltpu.sync_copy(
            acc.at[pl.ds(_NS * _RPT, _RREM)],
            out.at[c].at[pl.ds(_NS * _RPT, _RREM)],
        )


_mp_kernel = functools.partial(
    pl.kernel,
    out_type=jax.ShapeDtypeStruct((_NC, _N, _HALF), jnp.float32),
    mesh=plsc.VectorSubcoreMesh(core_axis_name="c", subcore_axis_name="s"),
    scratch_types=[
        pltpu.VMEM((_CHUNK,), jnp.int32),
        pltpu.VMEM((_CHUNK,), jnp.int32),
        pltpu.VMEM((_CHUNK, _HALF), jnp.float32),
        pltpu.VMEM_SHARED((_N, _HALF), jnp.float32),
        pltpu.SemaphoreType.DMA,
    ],
    compiler_params=pltpu.CompilerParams(use_tc_tiling_on_sc=False),
)(_sc_body)


def kernel(x, edge_index):
    x2 = jnp.concatenate([x[:, :_HALF], x[:, _HALF:]], axis=0)
    src = edge_index[0]
    dst = edge_index[1]
    zero = jnp.zeros((_N, _HALF), jnp.float32)
    out3 = _mp_kernel(x2, src, dst, zero)
    return out3.transpose(1, 0, 2).reshape(_N, _D)


# speedup vs baseline: 6.3695x; 1.8477x over previous
import functools

import jax
import jax.numpy as jnp
from jax import lax
from jax.experimental import pallas as pl
from jax.experimental.pallas import tpu as pltpu
from jax.experimental.pallas import tpu_sc as plsc

_N = 10000
_E = 320000
_D = 128
_NC = 2
_NS = 16
_HALF = _D // _NC
_EPT = _E // _NS
_CHUNK = 80
_NCHUNK = _EPT // _CHUNK
_RPT = 624
_RREM = _N - _NS * _RPT


def _sc_body(x2, srcs, dsts, zero, out, sidx, didx, rows, acc, gsem0, gsem1):
    c = lax.axis_index("c")
    s = lax.axis_index("s")
    r0 = s * _RPT

    pltpu.sync_copy(zero.at[pl.ds(r0, _RPT)], acc.at[pl.ds(r0, _RPT)])

    @pl.when(s == 0)
    def _zero_rem():
        pltpu.sync_copy(
            zero.at[pl.ds(_NS * _RPT, _RREM)], acc.at[pl.ds(_NS * _RPT, _RREM)]
        )

    plsc.subcore_barrier()

    pltpu.sync_copy(srcs.at[c, s], sidx)
    pltpu.sync_copy(dsts.at[s], didx)

    gsem = (gsem0, gsem1)
    pltpu.async_copy(x2.at[sidx.at[0]], rows.at[0], gsem[0])

    @pl.loop(0, _NCHUNK, step=2)
    def _chunks(i):
        for b in range(2):
            k = i + b
            pltpu.make_async_copy(x2.at[sidx.at[k]], rows.at[b], gsem[b]).wait()
            if b == 0:
                pltpu.async_copy(x2.at[sidx.at[k + 1]], rows.at[1 - b], gsem[1 - b])
            else:

                @pl.when(k + 1 < _NCHUNK)
                def _prefetch():
                    pltpu.async_copy(
                        x2.at[sidx.at[k + 1]], rows.at[1 - b], gsem[1 - b]
                    )

            pltpu.sync_copy(rows.at[b], acc.at[didx.at[k]], add=True)

    plsc.subcore_barrier()

    pltpu.sync_copy(acc.at[pl.ds(r0, _RPT)], out.at[c].at[pl.ds(r0, _RPT)])

    @pl.when(s == 0)
    def _write_rem():
        pltpu.sync_copy(
            acc.at[pl.ds(_NS * _RPT, _RREM)],
            out.at[c].at[pl.ds(_NS * _RPT, _RREM)],
        )


_mp_kernel = functools.partial(
    pl.kernel,
    out_type=jax.ShapeDtypeStruct((_NC, _N, _HALF), jnp.float32),
    mesh=plsc.VectorSubcoreMesh(core_axis_name="c", subcore_axis_name="s"),
    scratch_types=[
        pltpu.VMEM((_NCHUNK, _CHUNK), jnp.int32),
        pltpu.VMEM((_NCHUNK, _CHUNK), jnp.int32),
        pltpu.VMEM((2, _CHUNK, _HALF), jnp.float32),
        pltpu.VMEM_SHARED((_N, _HALF), jnp.float32),
        pltpu.SemaphoreType.DMA,
        pltpu.SemaphoreType.DMA,
    ],
    compiler_params=pltpu.CompilerParams(use_tc_tiling_on_sc=False),
)(_sc_body)


def kernel(x, edge_index):
    x2 = jnp.concatenate([x[:, :_HALF], x[:, _HALF:]], axis=0)
    src = edge_index[0]
    srcs = jnp.stack([src, src + _N]).reshape(_NC, _NS, _NCHUNK, _CHUNK)
    dsts = edge_index[1].reshape(_NS, _NCHUNK, _CHUNK)
    zero = jnp.zeros((_N, _HALF), jnp.float32)
    out3 = _mp_kernel(x2, srcs, dsts, zero)
    return out3.transpose(1, 0, 2).reshape(_N, _D)


# speedup vs baseline: 10.2434x; 1.6082x over previous
import functools

import jax
import jax.numpy as jnp
from jax import lax
from jax.experimental import pallas as pl
from jax.experimental.pallas import tpu as pltpu
from jax.experimental.pallas import tpu_sc as plsc

_N = 10000
_E = 320000
_D = 128
_NC = 2
_NS = 16
_HALF = _D // _NC
_EPT = _E // _NS
_CHUNK = 80
_NCHUNK = _EPT // _CHUNK
_RPT = 624
_RREM = _N - _NS * _RPT


_NB = 4
_AHEAD = 2
_BODY = _NCHUNK - _AHEAD


def _sc_body(x2, srcs, dsts, zero, out, sidx, didx, rows, acc, *sems):
    gsem = sems[:_NB]
    ssem = sems[_NB:]
    c = lax.axis_index("c")
    s = lax.axis_index("s")
    r0 = s * _RPT

    pltpu.sync_copy(zero.at[pl.ds(r0, _RPT)], acc.at[pl.ds(r0, _RPT)])

    @pl.when(s == 0)
    def _zero_rem():
        pltpu.sync_copy(
            zero.at[pl.ds(_NS * _RPT, _RREM)], acc.at[pl.ds(_NS * _RPT, _RREM)]
        )

    plsc.subcore_barrier()

    pltpu.sync_copy(srcs.at[c, s], sidx)
    pltpu.sync_copy(dsts.at[s], didx)

    def gather(k, b):
        pltpu.async_copy(x2.at[sidx.at[k]], rows.at[b], gsem[b])

    def gather_wait(k, b):
        pltpu.make_async_copy(x2.at[sidx.at[k]], rows.at[b], gsem[b]).wait()

    def scatter(k, b):
        pltpu.async_copy(rows.at[b], acc.at[didx.at[k]], ssem[b], add=True)

    def scatter_drain(b):
        pltpu.make_async_copy(zero.at[pl.ds(0, _CHUNK)], rows.at[b], ssem[b]).wait()

    for k in range(_AHEAD):
        gather(k, k % _NB)

    @pl.loop(0, _BODY, step=_NB)
    def _chunks(i):
        for b in range(_NB):
            k = i + b
            bg = (b + _AHEAD) % _NB
            if b < _AHEAD:
                @pl.when(k >= _AHEAD)
                def _drain():
                    scatter_drain(bg)

            else:
                scatter_drain(bg)
            gather(k + _AHEAD, bg)
            gather_wait(k, b)
            scatter(k, b)

    for k in range(_BODY, _NCHUNK):
        b = k % _NB
        gather_wait(k, b)
        scatter(k, b)

    for k in range(_NCHUNK - _NB, _NCHUNK):
        scatter_drain(k % _NB)

    plsc.subcore_barrier()

    pltpu.sync_copy(acc.at[pl.ds(r0, _RPT)], out.at[c].at[pl.ds(r0, _RPT)])

    @pl.when(s == 0)
    def _write_rem():
        pltpu.sync_copy(
            acc.at[pl.ds(_NS * _RPT, _RREM)],
            out.at[c].at[pl.ds(_NS * _RPT, _RREM)],
        )


_mp_kernel = functools.partial(
    pl.kernel,
    out_type=jax.ShapeDtypeStruct((_NC, _N, _HALF), jnp.float32),
    mesh=plsc.VectorSubcoreMesh(core_axis_name="c", subcore_axis_name="s"),
    scratch_types=[
        pltpu.VMEM((_NCHUNK, _CHUNK), jnp.int32),
        pltpu.VMEM((_NCHUNK, _CHUNK), jnp.int32),
        pltpu.VMEM((_NB, _CHUNK, _HALF), jnp.float32),
        pltpu.VMEM_SHARED((_N, _HALF), jnp.float32),
    ]
    + [pltpu.SemaphoreType.DMA] * (2 * _NB),
    compiler_params=pltpu.CompilerParams(use_tc_tiling_on_sc=False),
)(_sc_body)


def kernel(x, edge_index):
    x2 = jnp.concatenate([x[:, :_HALF], x[:, _HALF:]], axis=0)
    src = edge_index[0]
    srcs = jnp.stack([src, src + _N]).reshape(_NC, _NS, _NCHUNK, _CHUNK)
    dsts = edge_index[1].reshape(_NS, _NCHUNK, _CHUNK)
    zero = jnp.zeros((_N, _HALF), jnp.float32)
    out3 = _mp_kernel(x2, srcs, dsts, zero)
    return out3.transpose(1, 0, 2).reshape(_N, _D)


# speedup vs baseline: 12.6801x; 1.2379x over previous
import functools

import jax
import jax.numpy as jnp
from jax import lax
from jax.experimental import pallas as pl
from jax.experimental.pallas import tpu as pltpu
from jax.experimental.pallas import tpu_sc as plsc

_N = 10000
_E = 320000
_D = 128
_NC = 2
_NS = 16
_HALF = _D // _NC
_EPT = _E // _NS
_CHUNK = 80
_NCHUNK = _EPT // _CHUNK
_RPT = 624
_RREM = _N - _NS * _RPT
_NB = 4
_AHEAD = 2
_BODY = _NCHUNK - _AHEAD
_ZROWS = _NB * _CHUNK


def _sc_body(x2, srcs, dsts, out, sidx, didx, rows, acc, *sems):
    gsem = sems[:_NB]
    ssem = sems[_NB:]
    c = lax.axis_index("c")
    s = lax.axis_index("s")
    r0 = s * _RPT
    xv = x2.at[c]

    zvec = jnp.zeros((16,), jnp.float32)

    @pl.loop(0, _ZROWS)
    def _zero_rows(i):
        for j in range(_HALF // 16):
            rows[i, pl.ds(j * 16, 16)] = zvec

    for h in range(2):
        pltpu.sync_copy(
            rows.at[pl.ds(0, _RPT // 2)],
            acc.at[pl.ds(r0 + h * (_RPT // 2), _RPT // 2)],
        )

    @pl.when(s == 0)
    def _zero_rem():
        pltpu.sync_copy(
            rows.at[pl.ds(0, _RREM)], acc.at[pl.ds(_NS * _RPT, _RREM)]
        )

    plsc.subcore_barrier()

    pltpu.sync_copy(srcs.at[s], sidx)
    pltpu.sync_copy(dsts.at[s], didx)

    def rbuf(b):
        return rows.at[pl.ds(b * _CHUNK, _CHUNK)]

    def gather(k, b):
        pltpu.async_copy(xv.at[sidx.at[k]], rbuf(b), gsem[b])

    def gather_wait(k, b):
        pltpu.make_async_copy(xv.at[sidx.at[k]], rbuf(b), gsem[b]).wait()

    def scatter(k, b):
        pltpu.async_copy(rbuf(b), acc.at[didx.at[k]], ssem[b], add=True)

    def scatter_drain(b):
        pltpu.make_async_copy(x2.at[0].at[pl.ds(0, _CHUNK)], rbuf(b), ssem[b]).wait()

    for k in range(_AHEAD):
        gather(k, k % _NB)

    @pl.loop(0, _BODY, step=_NB)
    def _chunks(i):
        for b in range(_NB):
            k = i + b
            bg = (b + _AHEAD) % _NB
            if b < _AHEAD:
                @pl.when(k >= _AHEAD)
                def _drain():
                    scatter_drain(bg)

            else:
                scatter_drain(bg)
            gather(k + _AHEAD, bg)
            gather_wait(k, b)
            scatter(k, b)

    for k in range(_BODY, _NCHUNK):
        b = k % _NB
        gather_wait(k, b)
        scatter(k, b)

    for k in range(_NCHUNK - _NB, _NCHUNK):
        scatter_drain(k % _NB)

    plsc.subcore_barrier()

    pltpu.sync_copy(
        acc.at[pl.ds(r0, _RPT)],
        out.at[pl.ds(r0, _RPT), pl.ds(c * _HALF, _HALF)],
    )

    @pl.when(s == 0)
    def _write_rem():
        pltpu.sync_copy(
            acc.at[pl.ds(_NS * _RPT, _RREM)],
            out.at[pl.ds(_NS * _RPT, _RREM), pl.ds(c * _HALF, _HALF)],
        )


_mp_kernel = functools.partial(
    pl.kernel,
    out_type=jax.ShapeDtypeStruct((_N, _D), jnp.float32),
    mesh=plsc.VectorSubcoreMesh(core_axis_name="c", subcore_axis_name="s"),
    scratch_types=[
        pltpu.VMEM((_NCHUNK, _CHUNK), jnp.int32),
        pltpu.VMEM((_NCHUNK, _CHUNK), jnp.int32),
        pltpu.VMEM((_ZROWS, _HALF), jnp.float32),
        pltpu.VMEM_SHARED((_N, _HALF), jnp.float32),
    ]
    + [pltpu.SemaphoreType.DMA] * (2 * _NB),
    compiler_params=pltpu.CompilerParams(use_tc_tiling_on_sc=False),
)(_sc_body)


def kernel(x, edge_index):
    x2 = jnp.concatenate([x[:, :_HALF], x[:, _HALF:]], axis=0)
    x2 = x2.reshape(_NC, _N, _HALF)
    srcs = edge_index[0].reshape(_NS, _NCHUNK, _CHUNK)
    dsts = edge_index[1].reshape(_NS, _NCHUNK, _CHUNK)
    return _mp_kernel(x2, srcs, dsts)
